# Initial kernel scaffold; baseline (speedup 1.0000x reference)
import jax
import jax.numpy as jnp
from jax.experimental import pallas as pl


def kernel(x, token_table, pos_table):
    raise NotImplementedError("write your pallas kernel here")


# speedup vs baseline: 1.4257x; 1.4257x over previous
import functools

import jax
import jax.numpy as jnp
from jax import lax
from jax.experimental import pallas as pl
from jax.experimental.pallas import tpu as pltpu
from jax.experimental.pallas import tpu_sc as plsc

B = 4096
T = 200
D = 32
NC = 2
NS = 16
NW = NC * NS
SEQ_PER_W = B // NW
CH = 8
ROWS = CH * T
NCHUNK = SEQ_PER_W // CH


def _sc_embed(xf, token_table, pos_table):
    mesh = plsc.VectorSubcoreMesh(core_axis_name="c", subcore_axis_name="s")

    @functools.partial(
        pl.kernel,
        mesh=mesh,
        out_type=jax.ShapeDtypeStruct((B * T, D), jnp.float32),
        scratch_types=[
            pltpu.VMEM((ROWS,), jnp.int32),
            pltpu.VMEM((ROWS, D), jnp.float32),
            pltpu.VMEM((T, D), jnp.float32),
            pltpu.SemaphoreType.DMA,
        ],
        compiler_params=pltpu.CompilerParams(use_tc_tiling_on_sc=False),
    )
    def body(x_hbm, tok_hbm, pos_hbm, out_hbm, idx_v, rows_v, pos_v, sem):
        wid = lax.axis_index("s") * NC + lax.axis_index("c")
        base = wid * (SEQ_PER_W * T)
        pltpu.sync_copy(pos_hbm, pos_v)

        def chunk_body(ci, carry):
            row0 = base + ci * ROWS
            pltpu.sync_copy(x_hbm.at[pl.ds(row0, ROWS)], idx_v)
            pltpu.async_copy(tok_hbm.at[idx_v], rows_v, sem).wait()

            def t_body(t, c2):
                pv0 = pos_v[t, pl.ds(0, 16)]
                pv1 = pos_v[t, pl.ds(16, 16)]
                for s in range(CH):
                    r = s * T + t
                    rows_v[r, pl.ds(0, 16)] += pv0
                    rows_v[r, pl.ds(16, 16)] += pv1
                return c2

            lax.fori_loop(0, T, t_body, 0)
            pltpu.sync_copy(rows_v, out_hbm.at[pl.ds(row0, ROWS)])
            return carry

        lax.fori_loop(0, NCHUNK, chunk_body, 0)

    return body(xf, token_table, pos_table)


def kernel(x, token_table, pos_table):
    xf = x.reshape(B * T)
    out = _sc_embed(xf, token_table, pos_table)
    return out.reshape(B, T, D)


# speedup vs baseline: 1.4925x; 1.0469x over previous
import functools

import jax
import jax.numpy as jnp
from jax import lax
from jax.experimental import pallas as pl
from jax.experimental.pallas import tpu as pltpu
from jax.experimental.pallas import tpu_sc as plsc

B = 4096
T = 200
D = 32
NC = 2
NS = 16
NW = NC * NS
SEQ_PER_W = B // NW
CH = 4
ROWS = CH * T
NCHUNK = SEQ_PER_W // CH
NBUF = 3


def _sc_embed(xf, token_table, pos_table):
    mesh = plsc.VectorSubcoreMesh(core_axis_name="c", subcore_axis_name="s")

    @functools.partial(
        pl.kernel,
        mesh=mesh,
        out_type=jax.ShapeDtypeStruct((B * T, D), jnp.float32),
        scratch_types=[
            pltpu.VMEM((SEQ_PER_W * T,), jnp.int32),
            pltpu.VMEM((NBUF, ROWS, D), jnp.float32),
            pltpu.VMEM((T, D), jnp.float32),
            pltpu.SemaphoreType.DMA((NBUF,)),
            pltpu.SemaphoreType.DMA((NBUF,)),
        ],
        compiler_params=pltpu.CompilerParams(use_tc_tiling_on_sc=False),
    )
    def body(x_hbm, tok_hbm, pos_hbm, out_hbm, idx_v, rows_v, pos_v, gsem, wsem):
        wid = lax.axis_index("s") * NC + lax.axis_index("c")
        base = wid * (SEQ_PER_W * T)
        pltpu.sync_copy(pos_hbm, pos_v)
        pltpu.sync_copy(x_hbm.at[pl.ds(base, SEQ_PER_W * T)], idx_v)

        def gather_start(ci, b):
            idx_slice = idx_v.at[pl.ds(ci * ROWS, ROWS)]
            return pltpu.async_copy(tok_hbm.at[idx_slice], rows_v.at[b],
                                    gsem.at[b])

        def write_start(ci, b):
            return pltpu.async_copy(rows_v.at[b],
                                    out_hbm.at[pl.ds(base + ci * ROWS, ROWS)],
                                    wsem.at[b])

        def add_chunk(b):
            def t_body(t, c):
                pv0 = pos_v[t, pl.ds(0, 16)]
                pv1 = pos_v[t, pl.ds(16, 16)]
                for s in range(CH):
                    rows_v[b, s * T + t, pl.ds(0, 16)] += pv0
                    rows_v[b, s * T + t, pl.ds(16, 16)] += pv1
                return c
            lax.fori_loop(0, T, t_body, 0)

        gathers = [None] * NBUF
        writes = [None] * NBUF
        gathers[0] = gather_start(0, 0)
        gathers[1] = gather_start(1, 1)
        for ci in range(NCHUNK):
            b = ci % NBUF
            gathers[b].wait()
            add_chunk(b)
            writes[b] = write_start(ci, b)
            nb = (ci + 2) % NBUF
            if ci + 2 < NCHUNK:
                if writes[nb] is not None:
                    writes[nb].wait()
                gathers[nb] = gather_start(ci + 2, nb)
        writes[(NCHUNK - 2) % NBUF].wait()
        writes[(NCHUNK - 1) % NBUF].wait()

    return body(xf, token_table, pos_table)


def kernel(x, token_table, pos_table):
    xf = x.reshape(B * T)
    out = _sc_embed(xf, token_table, pos_table)
    return out.reshape(B, T, D)
